# baseline (speedup 1.0000x reference)
import jax
import jax.numpy as jnp
from jax import lax
from jax.experimental import pallas as pl
from jax.experimental.pallas import tpu as pltpu
from jax.experimental.pallas import tpu_sc as plsc

_N_CORES = 2
_N_SUBCORES = 16
_N_WORKERS = _N_CORES * _N_SUBCORES
_ROWS = 8
_LANES = 16
_D = 8


def _sc_body(idx_hbm, table_hbm, out_hbm, table_v, idx_v0, idx_v1, out_v0,
             out_v1, sem_i0, sem_i1, sem_o0, sem_o1):
    wid = lax.axis_index("s") * _N_CORES + lax.axis_index("c")
    b, s = idx_hbm.shape
    rows_per_w = b // _N_WORKERS
    row0 = wid * rows_per_w
    n_chunks = rows_per_w // _ROWS
    chunk_elems = _ROWS * s

    pltpu.sync_copy(table_hbm, table_v)

    idx_bufs = [idx_v0, idx_v1]
    out_bufs = [out_v0, out_v1]
    idx_sems = [sem_i0, sem_i1]
    out_sems = [sem_o0, sem_o1]

    lane = lax.iota(jnp.int32, _LANES)
    zero = lane * 0

    def compute_chunk(slot):
        idx_buf = idx_bufs[slot]
        out_buf = out_bufs[slot]

        def body(j, carry):
            t = lane + j * _LANES
            idxv = plsc.load_gather(idx_buf, [zero, t])
            addr = idxv * _D
            for k in range(_D):
                vals = plsc.load_gather(table_v, [addr + k])
                plsc.store_scatter(out_buf, [zero, t, zero + k], vals)
            return carry

        lax.fori_loop(0, chunk_elems // _LANES, body, 0, unroll=4)

    idx_copies = [None, None]
    out_copies = [None, None]
    for i in range(min(2, n_chunks)):
        idx_copies[i] = pltpu.async_copy(
            idx_hbm.at[pl.ds(row0 + i * _ROWS, _ROWS)], idx_bufs[i],
            idx_sems[i])

    for i in range(n_chunks):
        slot = i % 2
        idx_copies[slot].wait()
        if out_copies[slot] is not None:
            out_copies[slot].wait()
        compute_chunk(slot)
        out_copies[slot] = pltpu.async_copy(
            out_bufs[slot], out_hbm.at[pl.ds(row0 + i * _ROWS, _ROWS)],
            out_sems[slot])
        if i + 2 < n_chunks:
            idx_copies[slot] = pltpu.async_copy(
                idx_hbm.at[pl.ds(row0 + (i + 2) * _ROWS, _ROWS)],
                idx_bufs[slot], idx_sems[slot])

    for c in out_copies:
        if c is not None:
            c.wait()


def kernel(inputs, table):
    b, s = inputs.shape
    d = table.shape[1]
    table_flat = table.reshape(d * d)
    mesh = plsc.VectorSubcoreMesh(core_axis_name="c", subcore_axis_name="s")
    k = pl.kernel(
        _sc_body,
        out_type=jax.ShapeDtypeStruct((b, s, d), jnp.float32),
        mesh=mesh,
        scratch_types=[
            pltpu.VMEM((d * d,), jnp.float32),
            pltpu.VMEM((_ROWS, s), jnp.int32),
            pltpu.VMEM((_ROWS, s), jnp.int32),
            pltpu.VMEM((_ROWS, s, d), jnp.float32),
            pltpu.VMEM((_ROWS, s, d), jnp.float32),
            pltpu.SemaphoreType.DMA,
            pltpu.SemaphoreType.DMA,
            pltpu.SemaphoreType.DMA,
            pltpu.SemaphoreType.DMA,
        ],
        compiler_params=pltpu.CompilerParams(
            needs_layout_passes=False, use_tc_tiling_on_sc=False),
    )
    return k(inputs, table_flat)


# speedup vs baseline: 5.4675x; 5.4675x over previous
import jax
import jax.numpy as jnp
from jax import lax
from jax.experimental import pallas as pl
from jax.experimental.pallas import tpu as pltpu
from jax.experimental.pallas import tpu_sc as plsc

_N_CORES = 2
_N_SUBCORES = 16
_N_WORKERS = _N_CORES * _N_SUBCORES
_LANES = 16
_D = 8
_BTILES = 4
_SLAB = 8
_BCOLS = _BTILES * 128


def _sc_body(idx_hbm, table_hbm, out_hbm, table_v, idx_v0, idx_v1, out_v0,
             out_v1, si0, si1, so0, so1):
    wid = lax.axis_index("s") * _N_CORES + lax.axis_index("c")
    n_s = idx_hbm.shape[0]
    n_slabs = n_s // _SLAB
    col0 = wid * _BCOLS
    bt0 = wid * _BTILES

    pltpu.sync_copy(table_hbm, table_v)

    idx_bufs = [idx_v0, idx_v1]
    out_bufs = [out_v0, out_v1]
    idx_sems = [si0, si1]
    out_sems = [so0, so1]

    def start_idx(slab, slot):
        return pltpu.async_copy(
            idx_hbm.at[pl.ds(slab * _SLAB, _SLAB), pl.ds(col0, _BCOLS)],
            idx_bufs[slot], idx_sems[slot])

    def start_out(slab, slot):
        return pltpu.async_copy(
            out_bufs[slot],
            out_hbm.at[pl.ds(slab * _SLAB, _SLAB), pl.ds(bt0, _BTILES)],
            out_sems[slot])

    def compute_slab(slot):
        idx_buf = idx_bufs[slot]
        out_buf = out_bufs[slot]

        def body(s_in, carry):
            for t in range(_BTILES):
                for cc in range(128 // _LANES):
                    idxv = idx_buf[s_in, pl.ds(t * 128 + cc * _LANES, _LANES)]
                    a = idxv * _D
                    for k in range(_D):
                        vals = plsc.load_gather(table_v, [a + k])
                        out_buf[s_in, t, k, pl.ds(cc * _LANES, _LANES)] = vals
            return carry

        lax.fori_loop(0, _SLAB, body, 0)

    start_idx(0, 0)
    start_idx(1, 1)
    for slab in (0, 1):
        slot = slab
        pltpu.make_async_copy(
            idx_hbm.at[pl.ds(slab * _SLAB, _SLAB), pl.ds(col0, _BCOLS)],
            idx_bufs[slot], idx_sems[slot]).wait()
        compute_slab(slot)
        start_out(slab, slot)
        start_idx(slab + 2, slot)

    def loop_body(p, carry):
        for q in range(2):
            slab = 2 * p + q
            pltpu.make_async_copy(
                idx_hbm.at[pl.ds(slab * _SLAB, _SLAB), pl.ds(col0, _BCOLS)],
                idx_bufs[q], idx_sems[q]).wait()
            pltpu.make_async_copy(
                out_bufs[q],
                out_hbm.at[pl.ds(slab * _SLAB, _SLAB), pl.ds(bt0, _BTILES)],
                out_sems[q]).wait()
            compute_slab(q)
            start_out(slab, q)

            @pl.when(slab + 2 < n_slabs)
            def _():
                start_idx(slab + 2, q)
        return carry

    lax.fori_loop(1, (n_slabs - 1) // 2, loop_body, 0)

    last = n_slabs - 1
    pltpu.make_async_copy(
        idx_hbm.at[pl.ds(last * _SLAB, _SLAB), pl.ds(col0, _BCOLS)],
        idx_bufs[0], idx_sems[0]).wait()
    pltpu.make_async_copy(
        out_bufs[0],
        out_hbm.at[pl.ds(last * _SLAB, _SLAB), pl.ds(bt0, _BTILES)],
        out_sems[0]).wait()
    compute_slab(0)
    start_out(last, 0)

    pltpu.make_async_copy(
        out_bufs[0],
        out_hbm.at[pl.ds(last * _SLAB, _SLAB), pl.ds(bt0, _BTILES)],
        out_sems[0]).wait()
    pltpu.make_async_copy(
        out_bufs[1],
        out_hbm.at[pl.ds((last - 1) * _SLAB, _SLAB), pl.ds(bt0, _BTILES)],
        out_sems[1]).wait()


def kernel(inputs, table):
    b, s = inputs.shape
    d = table.shape[1]
    idx_t = inputs.T
    table_flat = table.reshape(d * d)
    mesh = plsc.VectorSubcoreMesh(core_axis_name="c", subcore_axis_name="s")
    k = pl.kernel(
        _sc_body,
        out_type=jax.ShapeDtypeStruct((s, b // 128, d, 128), jnp.float32),
        mesh=mesh,
        scratch_types=[
            pltpu.VMEM((d * d,), jnp.float32),
            pltpu.VMEM((_SLAB, _BCOLS), jnp.int32),
            pltpu.VMEM((_SLAB, _BCOLS), jnp.int32),
            pltpu.VMEM((_SLAB, _BTILES, _D, 128), jnp.float32),
            pltpu.VMEM((_SLAB, _BTILES, _D, 128), jnp.float32),
            pltpu.SemaphoreType.DMA,
            pltpu.SemaphoreType.DMA,
            pltpu.SemaphoreType.DMA,
            pltpu.SemaphoreType.DMA,
        ],
        compiler_params=pltpu.CompilerParams(
            needs_layout_passes=False, use_tc_tiling_on_sc=False),
    )
    out4 = k(idx_t, table_flat)
    return out4.transpose(1, 3, 0, 2).reshape(b, s, d)


# speedup vs baseline: 8.4802x; 1.5510x over previous
import jax
import jax.numpy as jnp
from jax import lax
from jax.experimental import pallas as pl
from jax.experimental.pallas import tpu as pltpu
from jax.experimental.pallas import tpu_sc as plsc

_N_CORES = 2
_N_SUBCORES = 16
_N_WORKERS = _N_CORES * _N_SUBCORES
_LANES = 16
_D = 8
_BTILES = 4
_SLAB = 8
_BCOLS = _BTILES * 128


def _sc_body(idx_hbm, table_hbm, out_hbm, table_v, idx_v0, idx_v1, out_v0,
             out_v1, si0, si1, so0, so1):
    wid = lax.axis_index("s") * _N_CORES + lax.axis_index("c")
    n_s = idx_hbm.shape[0]
    n_slabs = n_s // _SLAB
    col0 = wid * _BCOLS
    bt0 = wid * _BTILES

    pltpu.sync_copy(table_hbm, table_v)

    idx_bufs = [idx_v0, idx_v1]
    out_bufs = [out_v0, out_v1]
    idx_sems = [si0, si1]
    out_sems = [so0, so1]

    def start_idx(slab, slot):
        return pltpu.async_copy(
            idx_hbm.at[pl.ds(slab * _SLAB, _SLAB), pl.ds(col0, _BCOLS)],
            idx_bufs[slot], idx_sems[slot])

    def start_out(slab, slot):
        return pltpu.async_copy(
            out_bufs[slot],
            out_hbm.at[pl.ds(slab * _SLAB, _SLAB), pl.ds(bt0, _BTILES)],
            out_sems[slot])

    lane = lax.iota(jnp.int32, _LANES)

    def compute_slab(slot):
        idx_buf = idx_bufs[slot]
        out_buf = out_bufs[slot]

        def body(s_in, carry):
            for t in range(_BTILES):
                for cc in range(128 // _LANES):
                    idxv = idx_buf[s_in, pl.ds(t * 128 + cc * _LANES, _LANES)]
                    a = idxv * (_D * _LANES) + lane
                    for k in range(_D):
                        vals = plsc.load_gather(table_v, [a + k * _LANES])
                        out_buf[s_in, t, k, pl.ds(cc * _LANES, _LANES)] = vals
            return carry

        lax.fori_loop(0, _SLAB, body, 0)

    start_idx(0, 0)
    start_idx(1, 1)
    for slab in (0, 1):
        slot = slab
        pltpu.make_async_copy(
            idx_hbm.at[pl.ds(slab * _SLAB, _SLAB), pl.ds(col0, _BCOLS)],
            idx_bufs[slot], idx_sems[slot]).wait()
        compute_slab(slot)
        start_out(slab, slot)
        start_idx(slab + 2, slot)

    def loop_body(p, carry):
        for q in range(2):
            slab = 2 * p + q
            pltpu.make_async_copy(
                idx_hbm.at[pl.ds(slab * _SLAB, _SLAB), pl.ds(col0, _BCOLS)],
                idx_bufs[q], idx_sems[q]).wait()
            pltpu.make_async_copy(
                out_bufs[q],
                out_hbm.at[pl.ds(slab * _SLAB, _SLAB), pl.ds(bt0, _BTILES)],
                out_sems[q]).wait()
            compute_slab(q)
            start_out(slab, q)

            @pl.when(slab + 2 < n_slabs)
            def _():
                start_idx(slab + 2, q)
        return carry

    lax.fori_loop(1, (n_slabs - 1) // 2, loop_body, 0)

    last = n_slabs - 1
    pltpu.make_async_copy(
        idx_hbm.at[pl.ds(last * _SLAB, _SLAB), pl.ds(col0, _BCOLS)],
        idx_bufs[0], idx_sems[0]).wait()
    pltpu.make_async_copy(
        out_bufs[0],
        out_hbm.at[pl.ds(last * _SLAB, _SLAB), pl.ds(bt0, _BTILES)],
        out_sems[0]).wait()
    compute_slab(0)
    start_out(last, 0)

    pltpu.make_async_copy(
        out_bufs[0],
        out_hbm.at[pl.ds(last * _SLAB, _SLAB), pl.ds(bt0, _BTILES)],
        out_sems[0]).wait()
    pltpu.make_async_copy(
        out_bufs[1],
        out_hbm.at[pl.ds((last - 1) * _SLAB, _SLAB), pl.ds(bt0, _BTILES)],
        out_sems[1]).wait()


def kernel(inputs, table):
    b, s = inputs.shape
    d = table.shape[1]
    idx_t = inputs.T
    table_bc = jnp.broadcast_to(table.reshape(d * d)[:, None], (d * d, 16)).reshape(-1)
    mesh = plsc.VectorSubcoreMesh(core_axis_name="c", subcore_axis_name="s")
    k = pl.kernel(
        _sc_body,
        out_type=jax.ShapeDtypeStruct((s, b // 128, d, 128), jnp.float32),
        mesh=mesh,
        scratch_types=[
            pltpu.VMEM((d * d * 16,), jnp.float32),
            pltpu.VMEM((_SLAB, _BCOLS), jnp.int32),
            pltpu.VMEM((_SLAB, _BCOLS), jnp.int32),
            pltpu.VMEM((_SLAB, _BTILES, _D, 128), jnp.float32),
            pltpu.VMEM((_SLAB, _BTILES, _D, 128), jnp.float32),
            pltpu.SemaphoreType.DMA,
            pltpu.SemaphoreType.DMA,
            pltpu.SemaphoreType.DMA,
            pltpu.SemaphoreType.DMA,
        ],
        compiler_params=pltpu.CompilerParams(
            needs_layout_passes=False, use_tc_tiling_on_sc=False),
    )
    out4 = k(idx_t, table_bc)
    return out4.transpose(1, 3, 0, 2).reshape(b, s, d)


# speedup vs baseline: 15.8803x; 1.8726x over previous
import jax
import jax.numpy as jnp
from jax import lax
from jax.experimental import pallas as pl
from jax.experimental.pallas import tpu as pltpu
from jax.experimental.pallas import tpu_sc as plsc

_N_CORES = 2
_N_SUBCORES = 16
_N_WORKERS = _N_CORES * _N_SUBCORES
_LANES = 16
_D = 8
_BTILES = 4
_SLAB = 8
_BCOLS = _BTILES * 128


def _sc_body(idx_hbm, table_hbm, out_hbm, table_v, idx_v0, idx_v1, out_v0,
             out_v1, si0, si1, so0, so1):
    wid = lax.axis_index("s") * _N_CORES + lax.axis_index("c")
    n_s = idx_hbm.shape[0]
    n_slabs = n_s // _SLAB
    col0 = wid * _BCOLS
    bt0 = wid * _BTILES

    pltpu.sync_copy(table_hbm, table_v)

    idx_bufs = [idx_v0, idx_v1]
    out_bufs = [out_v0, out_v1]
    idx_sems = [si0, si1]
    out_sems = [so0, so1]

    def start_idx(slab, slot):
        return pltpu.async_copy(
            idx_hbm.at[pl.ds(slab * _SLAB, _SLAB), pl.ds(col0, _BCOLS)],
            idx_bufs[slot], idx_sems[slot])

    def start_out(slab, slot):
        return pltpu.async_copy(
            out_bufs[slot],
            out_hbm.at[pl.ds(slab * _SLAB, _SLAB), pl.ds(bt0, _BTILES)],
            out_sems[slot])

    lane = lax.iota(jnp.int32, _LANES)

    def compute_slab(slot):
        idx_buf = idx_bufs[slot]
        out_buf = out_bufs[slot]

        def body(s_in, carry):
            for t in range(_BTILES):
                for cc in range(128 // _LANES):
                    idxv = idx_buf[s_in, pl.ds(t * 128 + cc * _LANES, _LANES)]
                    a = idxv * (_D * _LANES) + lane
                    vals = [
                        plsc.load_gather(table_v, [a + k * _LANES])
                        for k in range(_D)
                    ]
                    for k in range(_D):
                        out_buf[s_in, t, k, pl.ds(cc * _LANES, _LANES)] = vals[k]
            return carry

        lax.fori_loop(0, _SLAB, body, 0)

    start_idx(0, 0)
    start_idx(1, 1)
    for slab in (0, 1):
        slot = slab
        pltpu.make_async_copy(
            idx_hbm.at[pl.ds(slab * _SLAB, _SLAB), pl.ds(col0, _BCOLS)],
            idx_bufs[slot], idx_sems[slot]).wait()
        compute_slab(slot)
        start_out(slab, slot)
        start_idx(slab + 2, slot)

    def loop_body(p, carry):
        for q in range(2):
            slab = 2 * p + q
            pltpu.make_async_copy(
                idx_hbm.at[pl.ds(slab * _SLAB, _SLAB), pl.ds(col0, _BCOLS)],
                idx_bufs[q], idx_sems[q]).wait()
            pltpu.make_async_copy(
                out_bufs[q],
                out_hbm.at[pl.ds(slab * _SLAB, _SLAB), pl.ds(bt0, _BTILES)],
                out_sems[q]).wait()
            compute_slab(q)
            start_out(slab, q)

            @pl.when(slab + 2 < n_slabs)
            def _():
                start_idx(slab + 2, q)
        return carry

    lax.fori_loop(1, (n_slabs - 1) // 2, loop_body, 0)

    last = n_slabs - 1
    pltpu.make_async_copy(
        idx_hbm.at[pl.ds(last * _SLAB, _SLAB), pl.ds(col0, _BCOLS)],
        idx_bufs[0], idx_sems[0]).wait()
    pltpu.make_async_copy(
        out_bufs[0],
        out_hbm.at[pl.ds(last * _SLAB, _SLAB), pl.ds(bt0, _BTILES)],
        out_sems[0]).wait()
    compute_slab(0)
    start_out(last, 0)

    pltpu.make_async_copy(
        out_bufs[0],
        out_hbm.at[pl.ds(last * _SLAB, _SLAB), pl.ds(bt0, _BTILES)],
        out_sems[0]).wait()
    pltpu.make_async_copy(
        out_bufs[1],
        out_hbm.at[pl.ds((last - 1) * _SLAB, _SLAB), pl.ds(bt0, _BTILES)],
        out_sems[1]).wait()


def kernel(inputs, table):
    b, s = inputs.shape
    d = table.shape[1]
    idx_t = inputs.T
    table_bc = jnp.broadcast_to(table.reshape(d * d)[:, None], (d * d, 16)).reshape(-1)
    mesh = plsc.VectorSubcoreMesh(core_axis_name="c", subcore_axis_name="s")
    k = pl.kernel(
        _sc_body,
        out_type=jax.ShapeDtypeStruct((s, b // 128, d, 128), jnp.float32),
        mesh=mesh,
        scratch_types=[
            pltpu.VMEM((d * d * 16,), jnp.float32),
            pltpu.VMEM((_SLAB, _BCOLS), jnp.int32),
            pltpu.VMEM((_SLAB, _BCOLS), jnp.int32),
            pltpu.VMEM((_SLAB, _BTILES, _D, 128), jnp.float32),
            pltpu.VMEM((_SLAB, _BTILES, _D, 128), jnp.float32),
            pltpu.SemaphoreType.DMA,
            pltpu.SemaphoreType.DMA,
            pltpu.SemaphoreType.DMA,
            pltpu.SemaphoreType.DMA,
        ],
        compiler_params=pltpu.CompilerParams(
            needs_layout_passes=False, use_tc_tiling_on_sc=False),
    )
    out4 = k(idx_t, table_bc)
    return out4.transpose(1, 3, 0, 2).reshape(b, s, d)
